# baseline (speedup 1.0000x reference)
import jax
import jax.numpy as jnp
from jax import lax
from jax.experimental import pallas as pl
from jax.experimental.pallas import tpu as pltpu
from jax.experimental.pallas import tpu_sc as plsc

_VOCAB = 1000000
_DIM = 64
_CTX = 200
_L = 16

_TAIL = 576
_ALIGNED = _VOCAB - _TAIL
_TC_BLK = 16384
_SC_M = 12
_SC_LEN = _SC_M * 32768
_TC_LEN = _ALIGNED - _SC_LEN
_TC_NB = _TC_LEN // _TC_BLK
_SC_V0 = _TC_LEN
_NW = 32
_WLEN = _SC_LEN // _NW
_WIN = 512
_NWIN = _WLEN // _WIN


def _gather_body(idx_ref, embT_hbm, s_ref, blocks_ref, sem):
    def _issue(j, c):
        tile = (idx_ref[j] // 128) * 128
        pltpu.make_async_copy(
            embT_hbm.at[:, pl.ds(tile, 128)],
            blocks_ref.at[j],
            sem,
        ).start()
        return c

    lax.fori_loop(0, _CTX, _issue, 0)

    def _wait(j, c):
        pltpu.make_async_copy(
            embT_hbm.at[:, pl.ds(0, 128)],
            blocks_ref.at[j],
            sem,
        ).wait()
        return c

    lax.fori_loop(0, _CTX, _wait, 0)

    lanes = lax.broadcasted_iota(jnp.int32, (_DIM, 128), 1)

    def _acc(j, acc):
        lane = idx_ref[j] % 128
        return acc + jnp.where(lanes == lane, blocks_ref[j], 0.0)

    acc = lax.fori_loop(0, _CTX, _acc, jnp.zeros((_DIM, 128), jnp.float32))
    s_ref[...] = jnp.sum(acc, axis=1, keepdims=True)


_DN = (((0,), (0,)), ((), ()))


def _tc_body(s_ref, w_ref, wt_ref, b_hbm, lo_ref, hi_ref, bscr, btscr, bsem):
    i = pl.program_id(0)

    @pl.when(i == 0)
    def _load_bias():
        pltpu.make_async_copy(b_hbm.at[pl.ds(0, _TC_LEN)], bscr, bsem).start()
        pltpu.make_async_copy(
            b_hbm.at[pl.ds(_ALIGNED, _TAIL)], btscr, bsem
        ).start()
        pltpu.make_async_copy(b_hbm.at[pl.ds(0, _TC_LEN)], bscr, bsem).wait()
        pltpu.make_async_copy(
            b_hbm.at[pl.ds(_ALIGNED, _TAIL)], btscr, bsem
        ).wait()

    s = s_ref[...]
    bias = jnp.reshape(bscr[pl.ds(i * _TC_BLK, _TC_BLK)], (1, _TC_BLK))
    lo_ref[...] = (
        lax.dot_general(s, w_ref[...], _DN, preferred_element_type=jnp.float32)
        + bias
    )

    @pl.when(i == _TC_NB - 1)
    def _tail():
        hi_ref[...] = (
            lax.dot_general(s, wt_ref[...], _DN, preferred_element_type=jnp.float32)
            + jnp.reshape(btscr[...], (1, _TAIL))
        )


def _sc_body(sb_hbm, w_hbm, b_hbm, out_hbm, sb_v, wb, bb, ob, wsem, bsem, osem):
    wid = lax.axis_index("s") * 2 + lax.axis_index("c")
    base = wid * _WLEN
    col0 = _SC_V0 + base
    pltpu.sync_copy(sb_hbm, sb_v)

    def _issue(t, k):
        pltpu.make_async_copy(
            w_hbm.at[:, pl.ds(col0 + t * _WIN, _WIN)], wb.at[k], wsem.at[k]
        ).start()
        pltpu.make_async_copy(
            b_hbm.at[pl.ds(col0 + t * _WIN, _WIN)], bb.at[k], bsem.at[k]
        ).start()

    _issue(0, 0)
    _issue(1, 1)

    @pl.loop(0, _NWIN, step=2)
    def _pair(t0):
        for kk in range(2):
            t = t0 + kk
            pltpu.make_async_copy(
                w_hbm.at[:, pl.ds(0, _WIN)], wb.at[kk], wsem.at[kk]
            ).wait()
            pltpu.make_async_copy(
                b_hbm.at[pl.ds(0, _WIN)], bb.at[kk], bsem.at[kk]
            ).wait()

            @pl.when(t >= 2)
            def _drain_out():
                pltpu.make_async_copy(
                    ob.at[kk], out_hbm.at[pl.ds(0, _WIN)], osem.at[kk]
                ).wait()

            @pl.loop(0, _WIN // _L)
            def _win(cc):
                co = cc * _L
                acc = bb[kk, pl.ds(co, _L)]
                for d in range(_DIM):
                    acc = acc + wb[kk, d, pl.ds(co, _L)] * sb_v[d, :]
                ob[kk, pl.ds(co, _L)] = acc

            pltpu.make_async_copy(
                ob.at[kk], out_hbm.at[pl.ds(base + t * _WIN, _WIN)], osem.at[kk]
            ).start()

            @pl.when(t + 2 < _NWIN)
            def _next():
                _issue(t + 2, kk)

    for kk in range(2):
        pltpu.make_async_copy(
            ob.at[kk], out_hbm.at[pl.ds(0, _WIN)], osem.at[kk]
        ).wait()


def kernel(context_idxs, emb_table, W, b):
    embT = jnp.transpose(emb_table)
    s = pl.pallas_call(
        _gather_body,
        grid=(1,),
        in_specs=[
            pl.BlockSpec(memory_space=pltpu.MemorySpace.SMEM),
            pl.BlockSpec(memory_space=pltpu.MemorySpace.HBM),
        ],
        out_specs=pl.BlockSpec((_DIM, 1), lambda i: (0, 0)),
        out_shape=jax.ShapeDtypeStruct((_DIM, 1), jnp.float32),
        scratch_shapes=[
            pltpu.VMEM((_CTX, _DIM, 128), jnp.float32),
            pltpu.SemaphoreType.DMA,
        ],
    )(context_idxs, embT)

    sb = jnp.broadcast_to(s, (_DIM, _L))
    w_tail = lax.slice(W, (0, _ALIGNED), (_DIM, _VOCAB))

    out_lo, out_hi = pl.pallas_call(
        _tc_body,
        grid=(_TC_NB,),
        in_specs=[
            pl.BlockSpec((_DIM, 1), lambda i: (0, 0)),
            pl.BlockSpec((_DIM, _TC_BLK), lambda i: (0, i)),
            pl.BlockSpec((_DIM, _TAIL), lambda i: (0, 0)),
            pl.BlockSpec(memory_space=pltpu.MemorySpace.HBM),
        ],
        out_specs=[
            pl.BlockSpec((1, _TC_BLK), lambda i: (0, i)),
            pl.BlockSpec((1, _TAIL), lambda i: (0, 0)),
        ],
        out_shape=[
            jax.ShapeDtypeStruct((1, _TC_LEN), jnp.float32),
            jax.ShapeDtypeStruct((1, _TAIL), jnp.float32),
        ],
        scratch_shapes=[
            pltpu.VMEM((_TC_LEN,), jnp.float32),
            pltpu.VMEM((_TAIL,), jnp.float32),
            pltpu.SemaphoreType.DMA,
        ],
    )(s, W, w_tail, b)

    mesh = plsc.VectorSubcoreMesh(core_axis_name="c", subcore_axis_name="s")
    sc_call = pl.kernel(
        _sc_body,
        out_type=jax.ShapeDtypeStruct((_SC_LEN,), jnp.float32),
        mesh=mesh,
        scratch_types=[
            pltpu.VMEM((_DIM, _L), jnp.float32),
            pltpu.VMEM((2, _DIM, _WIN), jnp.float32),
            pltpu.VMEM((2, _WIN), jnp.float32),
            pltpu.VMEM((2, _WIN), jnp.float32),
            pltpu.SemaphoreType.DMA((2,)),
            pltpu.SemaphoreType.DMA((2,)),
            pltpu.SemaphoreType.DMA((2,)),
        ],
        compiler_params=pltpu.CompilerParams(use_tc_tiling_on_sc=True),
    )
    out_sc = sc_call(sb, W, b)

    return jnp.concatenate(
        [out_lo, out_sc.reshape(1, _SC_LEN), out_hi], axis=1
    )


# speedup vs baseline: 1.0168x; 1.0168x over previous
import jax
import jax.numpy as jnp
from jax import lax
from jax.experimental import pallas as pl
from jax.experimental.pallas import tpu as pltpu
from jax.experimental.pallas import tpu_sc as plsc

_VOCAB = 1000000
_DIM = 64
_CTX = 200
_L = 16

_TAIL = 576
_ALIGNED = _VOCAB - _TAIL
_TC_BLK = 16384
_SC_M = 13
_SC_LEN = _SC_M * 32768
_TC_LEN = _ALIGNED - _SC_LEN
_TC_NB = _TC_LEN // _TC_BLK
_SC_V0 = _TC_LEN
_NW = 32
_WLEN = _SC_LEN // _NW
_WIN = 512
_NWIN = _WLEN // _WIN


def _gather_body(idx_ref, embT_hbm, s_ref, blocks_ref, sem):
    def _issue(j, c):
        tile = (idx_ref[j] // 128) * 128
        pltpu.make_async_copy(
            embT_hbm.at[:, pl.ds(tile, 128)],
            blocks_ref.at[j],
            sem,
        ).start()
        return c

    lax.fori_loop(0, _CTX, _issue, 0)

    def _wait(j, c):
        pltpu.make_async_copy(
            embT_hbm.at[:, pl.ds(0, 128)],
            blocks_ref.at[j],
            sem,
        ).wait()
        return c

    lax.fori_loop(0, _CTX, _wait, 0)

    lanes = lax.broadcasted_iota(jnp.int32, (_DIM, 128), 1)

    def _acc(j, acc):
        lane = idx_ref[j] % 128
        return acc + jnp.where(lanes == lane, blocks_ref[j], 0.0)

    acc = lax.fori_loop(0, _CTX, _acc, jnp.zeros((_DIM, 128), jnp.float32))
    s_ref[...] = jnp.sum(acc, axis=1, keepdims=True)


_DN = (((0,), (0,)), ((), ()))


def _tc_body(s_ref, w_ref, wt_ref, b_hbm, lo_ref, hi_ref, bscr, btscr, bsem):
    i = pl.program_id(0)

    @pl.when(i == 0)
    def _load_bias():
        pltpu.make_async_copy(b_hbm.at[pl.ds(0, _TC_LEN)], bscr, bsem).start()
        pltpu.make_async_copy(
            b_hbm.at[pl.ds(_ALIGNED, _TAIL)], btscr, bsem
        ).start()
        pltpu.make_async_copy(b_hbm.at[pl.ds(0, _TC_LEN)], bscr, bsem).wait()
        pltpu.make_async_copy(
            b_hbm.at[pl.ds(_ALIGNED, _TAIL)], btscr, bsem
        ).wait()

    s = s_ref[...]
    bias = jnp.reshape(bscr[pl.ds(i * _TC_BLK, _TC_BLK)], (1, _TC_BLK))
    lo_ref[...] = (
        lax.dot_general(s, w_ref[...], _DN, preferred_element_type=jnp.float32)
        + bias
    )

    @pl.when(i == _TC_NB - 1)
    def _tail():
        hi_ref[...] = (
            lax.dot_general(s, wt_ref[...], _DN, preferred_element_type=jnp.float32)
            + jnp.reshape(btscr[...], (1, _TAIL))
        )


def _sc_body(sb_hbm, w_hbm, b_hbm, out_hbm, sb_v, wb, bb, ob, wsem, bsem, osem):
    wid = lax.axis_index("s") * 2 + lax.axis_index("c")
    base = wid * _WLEN
    col0 = _SC_V0 + base
    pltpu.sync_copy(sb_hbm, sb_v)

    def _issue(t, k):
        pltpu.make_async_copy(
            w_hbm.at[:, pl.ds(col0 + t * _WIN, _WIN)], wb.at[k], wsem.at[k]
        ).start()
        pltpu.make_async_copy(
            b_hbm.at[pl.ds(col0 + t * _WIN, _WIN)], bb.at[k], bsem.at[k]
        ).start()

    _issue(0, 0)
    _issue(1, 1)

    @pl.loop(0, _NWIN, step=2)
    def _pair(t0):
        for kk in range(2):
            t = t0 + kk
            pltpu.make_async_copy(
                w_hbm.at[:, pl.ds(0, _WIN)], wb.at[kk], wsem.at[kk]
            ).wait()
            pltpu.make_async_copy(
                b_hbm.at[pl.ds(0, _WIN)], bb.at[kk], bsem.at[kk]
            ).wait()

            @pl.when(t >= 2)
            def _drain_out():
                pltpu.make_async_copy(
                    ob.at[kk], out_hbm.at[pl.ds(0, _WIN)], osem.at[kk]
                ).wait()

            @pl.loop(0, _WIN // _L)
            def _win(cc):
                co = cc * _L
                acc = bb[kk, pl.ds(co, _L)]
                for d in range(_DIM):
                    acc = acc + wb[kk, d, pl.ds(co, _L)] * sb_v[d, :]
                ob[kk, pl.ds(co, _L)] = acc

            pltpu.make_async_copy(
                ob.at[kk], out_hbm.at[pl.ds(base + t * _WIN, _WIN)], osem.at[kk]
            ).start()

            @pl.when(t + 2 < _NWIN)
            def _next():
                _issue(t + 2, kk)

    for kk in range(2):
        pltpu.make_async_copy(
            ob.at[kk], out_hbm.at[pl.ds(0, _WIN)], osem.at[kk]
        ).wait()


def kernel(context_idxs, emb_table, W, b):
    embT = jnp.transpose(emb_table)
    s = pl.pallas_call(
        _gather_body,
        grid=(1,),
        in_specs=[
            pl.BlockSpec(memory_space=pltpu.MemorySpace.SMEM),
            pl.BlockSpec(memory_space=pltpu.MemorySpace.HBM),
        ],
        out_specs=pl.BlockSpec((_DIM, 1), lambda i: (0, 0)),
        out_shape=jax.ShapeDtypeStruct((_DIM, 1), jnp.float32),
        scratch_shapes=[
            pltpu.VMEM((_CTX, _DIM, 128), jnp.float32),
            pltpu.SemaphoreType.DMA,
        ],
    )(context_idxs, embT)

    sb = jnp.broadcast_to(s, (_DIM, _L))
    w_tail = lax.slice(W, (0, _ALIGNED), (_DIM, _VOCAB))

    out_lo, out_hi = pl.pallas_call(
        _tc_body,
        grid=(_TC_NB,),
        in_specs=[
            pl.BlockSpec((_DIM, 1), lambda i: (0, 0)),
            pl.BlockSpec((_DIM, _TC_BLK), lambda i: (0, i)),
            pl.BlockSpec((_DIM, _TAIL), lambda i: (0, 0)),
            pl.BlockSpec(memory_space=pltpu.MemorySpace.HBM),
        ],
        out_specs=[
            pl.BlockSpec((1, _TC_BLK), lambda i: (0, i)),
            pl.BlockSpec((1, _TAIL), lambda i: (0, 0)),
        ],
        out_shape=[
            jax.ShapeDtypeStruct((1, _TC_LEN), jnp.float32),
            jax.ShapeDtypeStruct((1, _TAIL), jnp.float32),
        ],
        scratch_shapes=[
            pltpu.VMEM((_TC_LEN,), jnp.float32),
            pltpu.VMEM((_TAIL,), jnp.float32),
            pltpu.SemaphoreType.DMA,
        ],
    )(s, W, w_tail, b)

    mesh = plsc.VectorSubcoreMesh(core_axis_name="c", subcore_axis_name="s")
    sc_call = pl.kernel(
        _sc_body,
        out_type=jax.ShapeDtypeStruct((_SC_LEN,), jnp.float32),
        mesh=mesh,
        scratch_types=[
            pltpu.VMEM((_DIM, _L), jnp.float32),
            pltpu.VMEM((2, _DIM, _WIN), jnp.float32),
            pltpu.VMEM((2, _WIN), jnp.float32),
            pltpu.VMEM((2, _WIN), jnp.float32),
            pltpu.SemaphoreType.DMA((2,)),
            pltpu.SemaphoreType.DMA((2,)),
            pltpu.SemaphoreType.DMA((2,)),
        ],
        compiler_params=pltpu.CompilerParams(use_tc_tiling_on_sc=True),
    )
    out_sc = sc_call(sb, W, b)

    return jnp.concatenate(
        [out_lo, out_sc.reshape(1, _SC_LEN), out_hi], axis=1
    )
